# Initial kernel scaffold; baseline (speedup 1.0000x reference)
import jax
import jax.numpy as jnp
from jax.experimental import pallas as pl


def kernel(x_seq, edge_index, env_features, time_features, params):
    raise NotImplementedError("write your pallas kernel here")


# speedup vs baseline: 8.0401x; 8.0401x over previous
import functools

import jax
import jax.numpy as jnp
from jax import lax
from jax.experimental import pallas as pl
from jax.experimental.pallas import tpu as pltpu
from jax.experimental.pallas import tpu_sc as plsc

F32 = jnp.float32
N = 10000
NP = 10240
D = 128
HH = 128
T = 12
E = 320000
EC = 80
CW = E // (32 * EC)
NB = NP // 1024
BR = 1024
BR2 = 512
EPS = 1e-5


def _deg_kernel(dst2d, ones16, zeros16):
    mesh = plsc.VectorSubcoreMesh(core_axis_name="c", subcore_axis_name="s")

    @functools.partial(
        pl.kernel,
        out_type=jax.ShapeDtypeStruct((2 * NP, 16), F32),
        mesh=mesh,
        compiler_params=pltpu.CompilerParams(use_tc_tiling_on_sc=False),
        scratch_types=[
            pltpu.VMEM((CW, EC), jnp.int32),
            pltpu.VMEM((EC, 16), F32),
            pltpu.VMEM_SHARED((NP, 16), F32),
        ],
    )
    def k(dst_hbm, ones_hbm, zeros_hbm, out_hbm, dst_v, ones_v, acc):
        c = lax.axis_index("c")
        s = lax.axis_index("s")
        w = c * 16 + s
        pltpu.sync_copy(dst_hbm.at[pl.ds(w * CW, CW)], dst_v)
        pltpu.sync_copy(ones_hbm, ones_v)
        for i in range(16):
            pltpu.sync_copy(zeros_hbm, acc.at[pl.ds(s * 640 + i * 40, 40)])
        plsc.subcore_barrier()

        def chunk(ci, carry):
            pltpu.sync_copy(ones_v, acc.at[dst_v.at[ci]], add=True)
            return carry

        lax.fori_loop(0, CW, chunk, 0)
        plsc.subcore_barrier()
        pltpu.sync_copy(acc.at[pl.ds(s * 640, 640)],
                        out_hbm.at[pl.ds(c * NP + s * 640, 640)])

    return k(dst2d, ones16, zeros16)


def _make_spmm(tb):
    mesh = plsc.VectorSubcoreMesh(core_axis_name="c", subcore_axis_name="s")

    @functools.partial(
        pl.kernel,
        out_type=jax.ShapeDtypeStruct((2 * tb * NP, 128), F32),
        mesh=mesh,
        compiler_params=pltpu.CompilerParams(use_tc_tiling_on_sc=False),
        scratch_types=[
            pltpu.VMEM((CW * EC,), jnp.int32),
            pltpu.VMEM((CW, EC), jnp.int32),
            pltpu.VMEM((EC,), jnp.int32),
            pltpu.VMEM((EC, 128), F32),
            pltpu.VMEM((128, 128), F32),
            pltpu.VMEM_SHARED((NP, 128), F32),
            pltpu.SemaphoreType.DMA,
        ],
    )
    def k(hp_hbm, src_hbm, dst_hbm, zeros_hbm, out_hbm,
          src_v, dst_v, idx_v, rows_v, zbuf, acc, sem):
        c = lax.axis_index("c")
        s = lax.axis_index("s")
        w = c * 16 + s
        pltpu.sync_copy(src_hbm.at[pl.ds(w * CW * EC, CW * EC)], src_v)
        pltpu.sync_copy(dst_hbm.at[pl.ds(w * CW, CW)], dst_v)
        pltpu.sync_copy(zeros_hbm, zbuf)

        def tbody(t, carry):
            for i in range(5):
                pltpu.sync_copy(zbuf, acc.at[pl.ds(s * 640 + i * 128, 128)])
            plsc.subcore_barrier()
            base = t * NP

            def chunk(ci, carry2):
                for j in range(EC // 16):
                    idx_v[pl.ds(j * 16, 16)] = (
                        src_v[pl.ds(ci * EC + j * 16, 16)] + base)
                pltpu.async_copy(hp_hbm.at[idx_v], rows_v, sem).wait()
                pltpu.sync_copy(rows_v, acc.at[dst_v.at[ci]], add=True)
                return carry2

            lax.fori_loop(0, CW, chunk, 0)
            plsc.subcore_barrier()
            pltpu.sync_copy(acc.at[pl.ds(s * 640, 640)],
                            out_hbm.at[pl.ds((c * tb + t) * NP + s * 640, 640)])
            return carry

        lax.fori_loop(0, tb, tbody, 0)

    return k


_SPMM = {1: _make_spmm(1), 12: _make_spmm(12)}


def _bn_leaky(x, ssum, sssum, g, be, a, count):
    m = ssum / count
    v = sssum / count - m * m
    inv = lax.rsqrt(v + EPS)
    y = (x - m) * inv * g + be
    return jnp.where(y > 0, y, a * y)


def _dinv_kernel(deg0, deg1):
    def body(d0, d1, o_ref):
        nb = pl.program_id(0)
        deg = d0[:, :1] + d1[:, :1] + 1.0
        rows = nb * BR + lax.broadcasted_iota(jnp.int32, (BR, 1), 0)
        o_ref[...] = jnp.where(rows < N, lax.rsqrt(deg), 0.0)

    return pl.pallas_call(
        body,
        grid=(NB,),
        in_specs=[pl.BlockSpec((BR, 16), lambda i: (i, 0)),
                  pl.BlockSpec((BR, 16), lambda i: (i, 0))],
        out_specs=pl.BlockSpec((BR, 1), lambda i: (i, 0)),
        out_shape=jax.ShapeDtypeStruct((NP, 1), F32),
    )(deg0, deg1)


def _conv_mm(x, ssum, sssum, g, be, a, w, dinv, tb):
    def body(x_ref, s_ref, ss_ref, g_ref, be_ref, a_ref, w_ref, di_ref, o_ref):
        y = _bn_leaky(x_ref[0], s_ref[0], ss_ref[0], g_ref[...],
                      be_ref[...], a_ref[...], float(N))
        h = jnp.dot(y, w_ref[...], preferred_element_type=F32)
        o_ref[0] = h * di_ref[...]

    return pl.pallas_call(
        body,
        grid=(tb, NB),
        in_specs=[
            pl.BlockSpec((1, BR, 128), lambda t, i: (t, i, 0)),
            pl.BlockSpec((1, 1, 128), lambda t, i: (t, 0, 0)),
            pl.BlockSpec((1, 1, 128), lambda t, i: (t, 0, 0)),
            pl.BlockSpec((1, 128), lambda t, i: (0, 0)),
            pl.BlockSpec((1, 128), lambda t, i: (0, 0)),
            pl.BlockSpec((1, 128), lambda t, i: (0, 0)),
            pl.BlockSpec((128, 128), lambda t, i: (0, 0)),
            pl.BlockSpec((BR, 1), lambda t, i: (i, 0)),
        ],
        out_specs=pl.BlockSpec((1, BR, 128), lambda t, i: (t, i, 0)),
        out_shape=jax.ShapeDtypeStruct((tb, NP, 128), F32),
    )(x, ssum, sssum, g, be, a, w, dinv)


def _stats(p0, p1, hp, dinv, tb):
    def body(p0_ref, p1_ref, h_ref, di_ref, s_ref, cs_ref, css_ref):
        nb = pl.program_id(1)
        sv = (p0_ref[0] + p1_ref[0] + h_ref[0]) * di_ref[...]
        s_ref[0] = sv
        cs = jnp.sum(sv, axis=0, keepdims=True)
        css = jnp.sum(sv * sv, axis=0, keepdims=True)

        @pl.when(nb == 0)
        def _():
            cs_ref[0] = cs
            css_ref[0] = css

        @pl.when(nb != 0)
        def _():
            cs_ref[0] += cs
            css_ref[0] += css

    return pl.pallas_call(
        body,
        grid=(tb, NB),
        in_specs=[
            pl.BlockSpec((1, BR, 128), lambda t, i: (t, i, 0)),
            pl.BlockSpec((1, BR, 128), lambda t, i: (t, i, 0)),
            pl.BlockSpec((1, BR, 128), lambda t, i: (t, i, 0)),
            pl.BlockSpec((BR, 1), lambda t, i: (i, 0)),
        ],
        out_specs=[
            pl.BlockSpec((1, BR, 128), lambda t, i: (t, i, 0)),
            pl.BlockSpec((1, 1, 128), lambda t, i: (t, 0, 0)),
            pl.BlockSpec((1, 1, 128), lambda t, i: (t, 0, 0)),
        ],
        out_shape=[
            jax.ShapeDtypeStruct((tb, NP, 128), F32),
            jax.ShapeDtypeStruct((tb, 1, 128), F32),
            jax.ShapeDtypeStruct((tb, 1, 128), F32),
        ],
    )(p0, p1, hp, dinv)


def _mlp(x, ssum, sssum, g, be, a, w1, bias, lg, lbe, w2, b2, tb, hd, out_d):
    def body(x_ref, s_ref, ss_ref, g_ref, be_ref, a_ref, w1_ref, b_ref,
             lg_ref, lbe_ref, w2_ref, b2_ref, o_ref):
        y = _bn_leaky(x_ref[0], s_ref[0], ss_ref[0], g_ref[...],
                      be_ref[...], a_ref[...], float(N))
        h = jnp.dot(y, w1_ref[...], preferred_element_type=F32) + b_ref[0]
        h = jnp.maximum(h, 0.0)
        mu = jnp.sum(h, axis=-1, keepdims=True) / hd
        var = jnp.sum(h * h, axis=-1, keepdims=True) / hd - mu * mu
        hn = (h - mu) * lax.rsqrt(var + EPS) * lg_ref[...] + lbe_ref[...]
        o_ref[0] = jnp.dot(hn, w2_ref[...],
                           preferred_element_type=F32) + b2_ref[...]

    hdp = w1.shape[1]
    return pl.pallas_call(
        body,
        grid=(tb, NB),
        in_specs=[
            pl.BlockSpec((1, BR, 128), lambda t, i: (t, i, 0)),
            pl.BlockSpec((1, 1, 128), lambda t, i: (t, 0, 0)),
            pl.BlockSpec((1, 1, 128), lambda t, i: (t, 0, 0)),
            pl.BlockSpec((1, 128), lambda t, i: (0, 0)),
            pl.BlockSpec((1, 128), lambda t, i: (0, 0)),
            pl.BlockSpec((1, 128), lambda t, i: (0, 0)),
            pl.BlockSpec((128, hdp), lambda t, i: (0, 0)),
            pl.BlockSpec((1, 1, hdp), lambda t, i: (t, 0, 0)),
            pl.BlockSpec((1, hdp), lambda t, i: (0, 0)),
            pl.BlockSpec((1, hdp), lambda t, i: (0, 0)),
            pl.BlockSpec((hdp, out_d), lambda t, i: (0, 0)),
            pl.BlockSpec((1, out_d), lambda t, i: (0, 0)),
        ],
        out_specs=pl.BlockSpec((1, BR, out_d), lambda t, i: (t, i, 0)),
        out_shape=jax.ShapeDtypeStruct((tb, NP, out_d), F32),
    )(x, ssum, sssum, g, be, a, w1, bias, lg, lbe, w2, b2)


def _prep(envf, tmf, ep, tp, w1f_env, w1f_tm, b1f, w2f, wih, b2f, bih, bhh):
    def small_mlp(x, w1, b1, g, be, w2, b2, hd):
        h = jnp.maximum(jnp.dot(x, w1, preferred_element_type=F32) + b1, 0.0)
        mu = jnp.sum(h, -1, keepdims=True) / hd
        var = jnp.sum(h * h, -1, keepdims=True) / hd - mu * mu
        hn = (h - mu) * lax.rsqrt(var + EPS) * g + be
        return jnp.dot(hn, w2, preferred_element_type=F32) + b2

    def body(envf_ref, tmf_ref, ew1, eb1, eg, ebe, ew2, eb2,
             tw1, tb1, tg, tbe, tw2, tb2,
             w1e_ref, w1t_ref, b1f_ref, w2f_ref, wih_ref, b2f_ref,
             bih_ref, bhh_ref, bias_ref, w2p_ref, b2p_ref):
        ev = small_mlp(envf_ref[...], ew1[...], eb1[...], eg[...], ebe[...],
                       ew2[...], eb2[...], 16.0)
        tv = small_mlp(tmf_ref[...], tw1[...], tb1[...], tg[...], tbe[...],
                       tw2[...], tb2[...], 8.0)
        bias_ref[...] = (b1f_ref[...]
                         + jnp.dot(ev, w1e_ref[...], preferred_element_type=F32)
                         + jnp.dot(tv, w1t_ref[...], preferred_element_type=F32))
        w2p_ref[...] = jnp.dot(w2f_ref[...], wih_ref[...],
                               preferred_element_type=F32)
        b2p_ref[...] = (jnp.dot(b2f_ref[...], wih_ref[...],
                                preferred_element_type=F32)
                        + bih_ref[...] + bhh_ref[...])

    args = [envf, tmf,
            ep['W1'], ep['b1'].reshape(1, -1), ep['g'].reshape(1, -1),
            ep['be'].reshape(1, -1), ep['W2'], ep['b2'].reshape(1, -1),
            tp['W1'], tp['b1'].reshape(1, -1), tp['g'].reshape(1, -1),
            tp['be'].reshape(1, -1), tp['W2'], tp['b2'].reshape(1, -1),
            w1f_env, w1f_tm, b1f, w2f, wih, b2f, bih, bhh]
    return pl.pallas_call(
        body,
        out_shape=[
            jax.ShapeDtypeStruct((T, 140), F32),
            jax.ShapeDtypeStruct((140, 512), F32),
            jax.ShapeDtypeStruct((1, 512), F32),
        ],
    )(*args)


def _lstm_heads(fih, h0, whh, w1h, b1h, w2h, b2h):
    def body(f_ref, h0_ref, whh_ref, w1h_ref, b1h_ref, w2h_ref, b2h_ref,
             o_ref):
        h = h0_ref[...]
        c = jnp.zeros_like(h)
        whh_v = whh_ref[...]
        cols = []
        for t in range(T):
            gt = f_ref[t] + jnp.dot(h, whh_v, preferred_element_type=F32)
            ig = jax.nn.sigmoid(gt[:, :HH])
            fg = jax.nn.sigmoid(gt[:, HH:2 * HH])
            gg = jnp.tanh(gt[:, 2 * HH:3 * HH])
            og = jax.nn.sigmoid(gt[:, 3 * HH:])
            c = fg * c + ig * gg
            h = og * jnp.tanh(c)
            z = jnp.maximum(
                jnp.dot(h, w1h_ref[t], preferred_element_type=F32)
                + b1h_ref[t], 0.0)
            col = jnp.dot(z, w2h_ref[t][:, None],
                          preferred_element_type=F32) + b2h_ref[t]
            cols.append(col)
        o_ref[...] = jnp.concatenate(cols, axis=1)

    nb2 = NP // BR2
    return pl.pallas_call(
        body,
        grid=(nb2,),
        in_specs=[
            pl.BlockSpec((T, BR2, 512), lambda i: (0, i, 0)),
            pl.BlockSpec((BR2, 128), lambda i: (i, 0)),
            pl.BlockSpec((128, 512), lambda i: (0, 0)),
            pl.BlockSpec((T, 128, 64), lambda i: (0, 0, 0)),
            pl.BlockSpec((T, 64), lambda i: (0, 0)),
            pl.BlockSpec((T, 64), lambda i: (0, 0)),
            pl.BlockSpec((T, 1), lambda i: (0, 0)),
        ],
        out_specs=pl.BlockSpec((BR2, T), lambda i: (i, 0)),
        out_shape=jax.ShapeDtypeStruct((NP, T), F32),
    )(fih, h0, whh, w1h, b1h, w2h, b2h)


def _gcn_stack(x, gp, src_flat, dst2d, zeros128, dinv, tb):
    ssum = jnp.zeros((tb, 1, 128), F32)
    sssum = jnp.full((tb, 1, 128), float(N) * (1.0 - EPS), F32)
    g = jnp.ones((1, 128), F32)
    be = jnp.zeros((1, 128), F32)
    a = jnp.ones((1, 128), F32)
    for i in ('1', '2', '3'):
        w = gp['W' + i]
        hp = _conv_mm(x, ssum, sssum, g, be, a, w, dinv, tb)
        p = _SPMM[tb](hp.reshape(tb * NP, 128), src_flat, dst2d, zeros128)
        p = p.reshape(2, tb, NP, 128)
        x, ssum, sssum = _stats(p[0], p[1], hp, dinv, tb)
        g = gp['g' + i].reshape(1, 128)
        be = gp['be' + i].reshape(1, 128)
        a = gp['a' + i].reshape(1, 128)
    return x, ssum, sssum, g, be, a


def kernel(x_seq, edge_index, env_features, time_features, params):
    xpad = jnp.pad(x_seq, ((0, 0), (0, NP - N), (0, 0)))
    src_flat = jnp.asarray(edge_index[0], jnp.int32)
    dst_flat = jnp.asarray(edge_index[1], jnp.int32)
    dst2d = dst_flat.reshape(E // EC, EC)
    ones16 = jnp.ones((EC, 16), F32)
    zeros16 = jnp.zeros((40, 16), F32)
    zeros128 = jnp.zeros((128, 128), F32)

    fp = params['fusion']
    lp = params['lstm']
    bias1, w2p, b2p = _prep(
        env_features, time_features, params['env'], params['time'],
        fp['W1'][128:144], fp['W1'][144:152], fp['b1'].reshape(1, -1),
        fp['W2'], lp['Wih'], fp['b2'].reshape(1, -1),
        lp['bih'].reshape(1, -1), lp['bhh'].reshape(1, -1))

    degp = _deg_kernel(dst2d, ones16, zeros16)
    dinv = _dinv_kernel(degp[:NP], degp[NP:])

    s3h, ssh, sssh, gh, beh, ah = _gcn_stack(
        xpad[0:1], params['gcn_h0'], src_flat, dst2d, zeros128, dinv, 1)
    s3q, ssq, sssq, gq, beq, aq = _gcn_stack(
        xpad[1:13], params['gcn_seq'], src_flat, dst2d, zeros128, dinv, 12)

    hp0 = params['h0c0']
    h_init = _mlp(
        s3h, ssh, sssh, gh, beh, ah,
        hp0['W1'], hp0['b1'].reshape(1, 1, 128),
        hp0['g'].reshape(1, 128), hp0['be'].reshape(1, 128),
        hp0['W2'], hp0['b2'].reshape(1, 128), 1, 128.0, 128)[0]

    w1f = jnp.pad(fp['W1'][:128], ((0, 0), (0, 116)))
    biasp = jnp.pad(bias1, ((0, 0), (0, 116))).reshape(T, 1, 256)
    lgf = jnp.pad(fp['g'], (0, 116)).reshape(1, 256)
    lbef = jnp.pad(fp['be'], (0, 116)).reshape(1, 256)
    w2pp = jnp.pad(w2p, ((0, 116), (0, 0)))
    fih = _mlp(s3q, ssq, sssq, gq, beq, aq,
               w1f, biasp, lgf, lbef, w2pp, b2p, 12, 140.0, 512)

    ph = params['heads']
    out_pad = _lstm_heads(
        fih, h_init, lp['Whh'], ph['W1'], ph['b1'],
        ph['W2'][:, :, 0], ph['b2'])
    return out_pad[:N]


# speedup vs baseline: 10.3263x; 1.2844x over previous
import functools

import jax
import jax.numpy as jnp
from jax import lax
from jax.experimental import pallas as pl
from jax.experimental.pallas import tpu as pltpu
from jax.experimental.pallas import tpu_sc as plsc

F32 = jnp.float32
N = 10000
NP = 10240
D = 128
HH = 128
T = 12
E = 320000
EC = 80
CW = E // (32 * EC)
NB = NP // 1024
BR = 1024
BR2 = 512
EPS = 1e-5


def _deg_kernel(dst2d, ones16, zeros16):
    mesh = plsc.VectorSubcoreMesh(core_axis_name="c", subcore_axis_name="s")

    @functools.partial(
        pl.kernel,
        out_type=jax.ShapeDtypeStruct((2 * NP, 16), F32),
        mesh=mesh,
        compiler_params=pltpu.CompilerParams(use_tc_tiling_on_sc=False),
        scratch_types=[
            pltpu.VMEM((CW, EC), jnp.int32),
            pltpu.VMEM((EC, 16), F32),
            pltpu.VMEM_SHARED((NP, 16), F32),
        ],
    )
    def k(dst_hbm, ones_hbm, zeros_hbm, out_hbm, dst_v, ones_v, acc):
        c = lax.axis_index("c")
        s = lax.axis_index("s")
        w = c * 16 + s
        pltpu.sync_copy(dst_hbm.at[pl.ds(w * CW, CW)], dst_v)
        pltpu.sync_copy(ones_hbm, ones_v)
        for i in range(16):
            pltpu.sync_copy(zeros_hbm, acc.at[pl.ds(s * 640 + i * 40, 40)])
        plsc.subcore_barrier()

        def chunk(ci, carry):
            pltpu.sync_copy(ones_v, acc.at[dst_v.at[ci]], add=True)
            return carry

        lax.fori_loop(0, CW, chunk, 0)
        plsc.subcore_barrier()
        pltpu.sync_copy(acc.at[pl.ds(s * 640, 640)],
                        out_hbm.at[pl.ds(c * NP + s * 640, 640)])

    return k(dst2d, ones16, zeros16)


def _make_spmm(tb):
    mesh = plsc.VectorSubcoreMesh(core_axis_name="c", subcore_axis_name="s")

    @functools.partial(
        pl.kernel,
        out_type=jax.ShapeDtypeStruct((2 * tb * NP, 128), F32),
        mesh=mesh,
        compiler_params=pltpu.CompilerParams(use_tc_tiling_on_sc=False),
        scratch_types=[
            pltpu.VMEM((CW * EC,), jnp.int32),
            pltpu.VMEM((CW, EC), jnp.int32),
            [pltpu.VMEM((EC,), jnp.int32) for _ in range(2)],
            [pltpu.VMEM((EC, 128), F32) for _ in range(2)],
            pltpu.VMEM_SHARED((NP, 128), F32),
            [pltpu.SemaphoreType.DMA for _ in range(2)],
            [pltpu.SemaphoreType.DMA for _ in range(2)],
        ],
    )
    def k(hp_hbm, src_hbm, dst_hbm, zeros_hbm, out_hbm,
          src_v, dst_v, idx, rows, acc, gsem, ssem):
        c = lax.axis_index("c")
        s = lax.axis_index("s")
        w = c * 16 + s
        pltpu.sync_copy(src_hbm.at[pl.ds(w * CW * EC, CW * EC)], src_v)
        pltpu.sync_copy(dst_hbm.at[pl.ds(w * CW, CW)], dst_v)

        def build_gather(b, cn, base):
            for j in range(EC // 16):
                idx[b][pl.ds(j * 16, 16)] = (
                    src_v[pl.ds(cn * EC + j * 16, 16)] + base)
            pltpu.async_copy(hp_hbm.at[idx[b]], rows[b], gsem[b])

        def wait_gather(b):
            pltpu.make_async_copy(hp_hbm.at[idx[b]], rows[b], gsem[b]).wait()

        def scatter(b, cn):
            pltpu.async_copy(rows[b], acc.at[dst_v.at[cn]], ssem[b], add=True)

        def wait_scatter(b):
            pltpu.make_async_copy(zeros_hbm.at[pl.ds(0, EC)], rows[b],
                                  ssem[b]).wait()

        def tbody(t, carry):
            pltpu.sync_copy(zeros_hbm, acc.at[pl.ds(s * 640, 640)])
            plsc.subcore_barrier()
            base = t * NP
            build_gather(0, 0, base)
            build_gather(1, 1, base)

            def body2(kk, carry2):
                cn = kk * 2
                wait_gather(0)
                scatter(0, cn)
                wait_gather(1)
                scatter(1, cn + 1)
                wait_scatter(0)
                build_gather(0, cn + 2, base)
                wait_scatter(1)

                @pl.when(cn + 3 < CW)
                def _():
                    build_gather(1, cn + 3, base)
                return carry2

            lax.fori_loop(0, (CW - 1) // 2, body2, 0)
            wait_gather(0)
            scatter(0, CW - 1)
            wait_scatter(0)
            plsc.subcore_barrier()
            pltpu.sync_copy(acc.at[pl.ds(s * 640, 640)],
                            out_hbm.at[pl.ds((c * tb + t) * NP + s * 640, 640)])
            return carry

        lax.fori_loop(0, tb, tbody, 0)

    return k


_SPMM = {1: _make_spmm(1), 12: _make_spmm(12)}


def _bn_leaky(x, ssum, sssum, g, be, a, count):
    m = ssum / count
    v = sssum / count - m * m
    inv = lax.rsqrt(v + EPS)
    y = (x - m) * inv * g + be
    return jnp.where(y > 0, y, a * y)


def _dinv_kernel(deg0, deg1):
    def body(d0, d1, o_ref):
        nb = pl.program_id(0)
        deg = d0[:, :1] + d1[:, :1] + 1.0
        rows = nb * BR + lax.broadcasted_iota(jnp.int32, (BR, 1), 0)
        o_ref[...] = jnp.where(rows < N, lax.rsqrt(deg), 0.0)

    return pl.pallas_call(
        body,
        grid=(NB,),
        in_specs=[pl.BlockSpec((BR, 16), lambda i: (i, 0)),
                  pl.BlockSpec((BR, 16), lambda i: (i, 0))],
        out_specs=pl.BlockSpec((BR, 1), lambda i: (i, 0)),
        out_shape=jax.ShapeDtypeStruct((NP, 1), F32),
    )(deg0, deg1)


def _conv_mm(x, ssum, sssum, g, be, a, w, dinv, tb):
    def body(x_ref, s_ref, ss_ref, g_ref, be_ref, a_ref, w_ref, di_ref, o_ref):
        y = _bn_leaky(x_ref[0], s_ref[0], ss_ref[0], g_ref[...],
                      be_ref[...], a_ref[...], float(N))
        h = jnp.dot(y, w_ref[...], preferred_element_type=F32)
        o_ref[0] = h * di_ref[...]

    return pl.pallas_call(
        body,
        grid=(tb, NB),
        in_specs=[
            pl.BlockSpec((1, BR, 128), lambda t, i: (t, i, 0)),
            pl.BlockSpec((1, 1, 128), lambda t, i: (t, 0, 0)),
            pl.BlockSpec((1, 1, 128), lambda t, i: (t, 0, 0)),
            pl.BlockSpec((1, 128), lambda t, i: (0, 0)),
            pl.BlockSpec((1, 128), lambda t, i: (0, 0)),
            pl.BlockSpec((1, 128), lambda t, i: (0, 0)),
            pl.BlockSpec((128, 128), lambda t, i: (0, 0)),
            pl.BlockSpec((BR, 1), lambda t, i: (i, 0)),
        ],
        out_specs=pl.BlockSpec((1, BR, 128), lambda t, i: (t, i, 0)),
        out_shape=jax.ShapeDtypeStruct((tb, NP, 128), F32),
    )(x, ssum, sssum, g, be, a, w, dinv)


def _stats(p0, p1, hp, dinv, tb):
    def body(p0_ref, p1_ref, h_ref, di_ref, s_ref, cs_ref, css_ref):
        nb = pl.program_id(1)
        sv = (p0_ref[0] + p1_ref[0] + h_ref[0]) * di_ref[...]
        s_ref[0] = sv
        cs = jnp.sum(sv, axis=0, keepdims=True)
        css = jnp.sum(sv * sv, axis=0, keepdims=True)

        @pl.when(nb == 0)
        def _():
            cs_ref[0] = cs
            css_ref[0] = css

        @pl.when(nb != 0)
        def _():
            cs_ref[0] += cs
            css_ref[0] += css

    return pl.pallas_call(
        body,
        grid=(tb, NB),
        in_specs=[
            pl.BlockSpec((1, BR, 128), lambda t, i: (t, i, 0)),
            pl.BlockSpec((1, BR, 128), lambda t, i: (t, i, 0)),
            pl.BlockSpec((1, BR, 128), lambda t, i: (t, i, 0)),
            pl.BlockSpec((BR, 1), lambda t, i: (i, 0)),
        ],
        out_specs=[
            pl.BlockSpec((1, BR, 128), lambda t, i: (t, i, 0)),
            pl.BlockSpec((1, 1, 128), lambda t, i: (t, 0, 0)),
            pl.BlockSpec((1, 1, 128), lambda t, i: (t, 0, 0)),
        ],
        out_shape=[
            jax.ShapeDtypeStruct((tb, NP, 128), F32),
            jax.ShapeDtypeStruct((tb, 1, 128), F32),
            jax.ShapeDtypeStruct((tb, 1, 128), F32),
        ],
    )(p0, p1, hp, dinv)


def _mlp(x, ssum, sssum, g, be, a, w1, bias, lg, lbe, w2, b2, tb, hd, out_d):
    def body(x_ref, s_ref, ss_ref, g_ref, be_ref, a_ref, w1_ref, b_ref,
             lg_ref, lbe_ref, w2_ref, b2_ref, o_ref):
        y = _bn_leaky(x_ref[0], s_ref[0], ss_ref[0], g_ref[...],
                      be_ref[...], a_ref[...], float(N))
        h = jnp.dot(y, w1_ref[...], preferred_element_type=F32) + b_ref[0]
        h = jnp.maximum(h, 0.0)
        mu = jnp.sum(h, axis=-1, keepdims=True) / hd
        var = jnp.sum(h * h, axis=-1, keepdims=True) / hd - mu * mu
        hn = (h - mu) * lax.rsqrt(var + EPS) * lg_ref[...] + lbe_ref[...]
        o_ref[0] = jnp.dot(hn, w2_ref[...],
                           preferred_element_type=F32) + b2_ref[...]

    hdp = w1.shape[1]
    return pl.pallas_call(
        body,
        grid=(tb, NB),
        in_specs=[
            pl.BlockSpec((1, BR, 128), lambda t, i: (t, i, 0)),
            pl.BlockSpec((1, 1, 128), lambda t, i: (t, 0, 0)),
            pl.BlockSpec((1, 1, 128), lambda t, i: (t, 0, 0)),
            pl.BlockSpec((1, 128), lambda t, i: (0, 0)),
            pl.BlockSpec((1, 128), lambda t, i: (0, 0)),
            pl.BlockSpec((1, 128), lambda t, i: (0, 0)),
            pl.BlockSpec((128, hdp), lambda t, i: (0, 0)),
            pl.BlockSpec((1, 1, hdp), lambda t, i: (t, 0, 0)),
            pl.BlockSpec((1, hdp), lambda t, i: (0, 0)),
            pl.BlockSpec((1, hdp), lambda t, i: (0, 0)),
            pl.BlockSpec((hdp, out_d), lambda t, i: (0, 0)),
            pl.BlockSpec((1, out_d), lambda t, i: (0, 0)),
        ],
        out_specs=pl.BlockSpec((1, BR, out_d), lambda t, i: (t, i, 0)),
        out_shape=jax.ShapeDtypeStruct((tb, NP, out_d), F32),
    )(x, ssum, sssum, g, be, a, w1, bias, lg, lbe, w2, b2)


def _prep(envf, tmf, ep, tp, w1f_env, w1f_tm, b1f, w2f, wih, b2f, bih, bhh):
    def small_mlp(x, w1, b1, g, be, w2, b2, hd):
        h = jnp.maximum(jnp.dot(x, w1, preferred_element_type=F32) + b1, 0.0)
        mu = jnp.sum(h, -1, keepdims=True) / hd
        var = jnp.sum(h * h, -1, keepdims=True) / hd - mu * mu
        hn = (h - mu) * lax.rsqrt(var + EPS) * g + be
        return jnp.dot(hn, w2, preferred_element_type=F32) + b2

    def body(envf_ref, tmf_ref, ew1, eb1, eg, ebe, ew2, eb2,
             tw1, tb1, tg, tbe, tw2, tb2,
             w1e_ref, w1t_ref, b1f_ref, w2f_ref, wih_ref, b2f_ref,
             bih_ref, bhh_ref, bias_ref, w2p_ref, b2p_ref):
        ev = small_mlp(envf_ref[...], ew1[...], eb1[...], eg[...], ebe[...],
                       ew2[...], eb2[...], 16.0)
        tv = small_mlp(tmf_ref[...], tw1[...], tb1[...], tg[...], tbe[...],
                       tw2[...], tb2[...], 8.0)
        bias_ref[...] = (b1f_ref[...]
                         + jnp.dot(ev, w1e_ref[...], preferred_element_type=F32)
                         + jnp.dot(tv, w1t_ref[...], preferred_element_type=F32))
        w2p_ref[...] = jnp.dot(w2f_ref[...], wih_ref[...],
                               preferred_element_type=F32)
        b2p_ref[...] = (jnp.dot(b2f_ref[...], wih_ref[...],
                                preferred_element_type=F32)
                        + bih_ref[...] + bhh_ref[...])

    args = [envf, tmf,
            ep['W1'], ep['b1'].reshape(1, -1), ep['g'].reshape(1, -1),
            ep['be'].reshape(1, -1), ep['W2'], ep['b2'].reshape(1, -1),
            tp['W1'], tp['b1'].reshape(1, -1), tp['g'].reshape(1, -1),
            tp['be'].reshape(1, -1), tp['W2'], tp['b2'].reshape(1, -1),
            w1f_env, w1f_tm, b1f, w2f, wih, b2f, bih, bhh]
    return pl.pallas_call(
        body,
        out_shape=[
            jax.ShapeDtypeStruct((T, 140), F32),
            jax.ShapeDtypeStruct((140, 512), F32),
            jax.ShapeDtypeStruct((1, 512), F32),
        ],
    )(*args)


def _lstm_heads(fih, h0, whh, w1h, b1h, w2h, b2h):
    def body(f_ref, h0_ref, whh_ref, w1h_ref, b1h_ref, w2h_ref, b2h_ref,
             o_ref):
        h = h0_ref[...]
        c = jnp.zeros_like(h)
        whh_v = whh_ref[...]
        cols = []
        for t in range(T):
            gt = f_ref[t] + jnp.dot(h, whh_v, preferred_element_type=F32)
            ig = jax.nn.sigmoid(gt[:, :HH])
            fg = jax.nn.sigmoid(gt[:, HH:2 * HH])
            gg = jnp.tanh(gt[:, 2 * HH:3 * HH])
            og = jax.nn.sigmoid(gt[:, 3 * HH:])
            c = fg * c + ig * gg
            h = og * jnp.tanh(c)
            z = jnp.maximum(
                jnp.dot(h, w1h_ref[t], preferred_element_type=F32)
                + b1h_ref[t], 0.0)
            col = jnp.dot(z, w2h_ref[t][:, None],
                          preferred_element_type=F32) + b2h_ref[t]
            cols.append(col)
        o_ref[...] = jnp.concatenate(cols, axis=1)

    nb2 = NP // BR2
    return pl.pallas_call(
        body,
        grid=(nb2,),
        in_specs=[
            pl.BlockSpec((T, BR2, 512), lambda i: (0, i, 0)),
            pl.BlockSpec((BR2, 128), lambda i: (i, 0)),
            pl.BlockSpec((128, 512), lambda i: (0, 0)),
            pl.BlockSpec((T, 128, 64), lambda i: (0, 0, 0)),
            pl.BlockSpec((T, 64), lambda i: (0, 0)),
            pl.BlockSpec((T, 64), lambda i: (0, 0)),
            pl.BlockSpec((T, 1), lambda i: (0, 0)),
        ],
        out_specs=pl.BlockSpec((BR2, T), lambda i: (i, 0)),
        out_shape=jax.ShapeDtypeStruct((NP, T), F32),
    )(fih, h0, whh, w1h, b1h, w2h, b2h)


def _gcn_stack(x, gp, src_flat, dst2d, zeros128, dinv, tb):
    ssum = jnp.zeros((tb, 1, 128), F32)
    sssum = jnp.full((tb, 1, 128), float(N) * (1.0 - EPS), F32)
    g = jnp.ones((1, 128), F32)
    be = jnp.zeros((1, 128), F32)
    a = jnp.ones((1, 128), F32)
    for i in ('1', '2', '3'):
        w = gp['W' + i]
        hp = _conv_mm(x, ssum, sssum, g, be, a, w, dinv, tb)
        p = _SPMM[tb](hp.reshape(tb * NP, 128), src_flat, dst2d, zeros128)
        p = p.reshape(2, tb, NP, 128)
        x, ssum, sssum = _stats(p[0], p[1], hp, dinv, tb)
        g = gp['g' + i].reshape(1, 128)
        be = gp['be' + i].reshape(1, 128)
        a = gp['a' + i].reshape(1, 128)
    return x, ssum, sssum, g, be, a


def kernel(x_seq, edge_index, env_features, time_features, params):
    xpad = jnp.pad(x_seq, ((0, 0), (0, NP - N), (0, 0)))
    src_flat = jnp.asarray(edge_index[0], jnp.int32)
    dst_flat = jnp.asarray(edge_index[1], jnp.int32)
    dst2d = dst_flat.reshape(E // EC, EC)
    ones16 = jnp.ones((EC, 16), F32)
    zeros16 = jnp.zeros((40, 16), F32)
    zeros128 = jnp.zeros((640, 128), F32)

    fp = params['fusion']
    lp = params['lstm']
    bias1, w2p, b2p = _prep(
        env_features, time_features, params['env'], params['time'],
        fp['W1'][128:144], fp['W1'][144:152], fp['b1'].reshape(1, -1),
        fp['W2'], lp['Wih'], fp['b2'].reshape(1, -1),
        lp['bih'].reshape(1, -1), lp['bhh'].reshape(1, -1))

    degp = _deg_kernel(dst2d, ones16, zeros16)
    dinv = _dinv_kernel(degp[:NP], degp[NP:])

    s3h, ssh, sssh, gh, beh, ah = _gcn_stack(
        xpad[0:1], params['gcn_h0'], src_flat, dst2d, zeros128, dinv, 1)
    s3q, ssq, sssq, gq, beq, aq = _gcn_stack(
        xpad[1:13], params['gcn_seq'], src_flat, dst2d, zeros128, dinv, 12)

    hp0 = params['h0c0']
    h_init = _mlp(
        s3h, ssh, sssh, gh, beh, ah,
        hp0['W1'], hp0['b1'].reshape(1, 1, 128),
        hp0['g'].reshape(1, 128), hp0['be'].reshape(1, 128),
        hp0['W2'], hp0['b2'].reshape(1, 128), 1, 128.0, 128)[0]

    w1f = jnp.pad(fp['W1'][:128], ((0, 0), (0, 116)))
    biasp = jnp.pad(bias1, ((0, 0), (0, 116))).reshape(T, 1, 256)
    lgf = jnp.pad(fp['g'], (0, 116)).reshape(1, 256)
    lbef = jnp.pad(fp['be'], (0, 116)).reshape(1, 256)
    w2pp = jnp.pad(w2p, ((0, 116), (0, 0)))
    fih = _mlp(s3q, ssq, sssq, gq, beq, aq,
               w1f, biasp, lgf, lbef, w2pp, b2p, 12, 140.0, 512)

    ph = params['heads']
    out_pad = _lstm_heads(
        fih, h_init, lp['Whh'], ph['W1'], ph['b1'],
        ph['W2'][:, :, 0], ph['b2'])
    return out_pad[:N]
